# baseline (speedup 1.0000x reference)
import jax
import jax.numpy as jnp
from jax.experimental import pallas as pl
from jax.experimental.pallas import tpu as pltpu

_TILE_M = 1024


def _ffn_body(x_ref, w1_ref, b1_ref, w2_ref, b2_ref, o_ref):
    xb = x_ref[...].astype(jnp.bfloat16)
    h = jnp.dot(xb, w1_ref[...], preferred_element_type=jnp.float32)
    h = jnp.maximum(h + b1_ref[...], 0.0).astype(jnp.bfloat16)
    y = jnp.dot(h, w2_ref[...], preferred_element_type=jnp.float32)
    o_ref[...] = (y + b2_ref[...]).astype(o_ref.dtype)


def _ffn_call(m_rows, tile_m, d_in, d_mid, d_out, out_dtype):
    const = lambda i: (0, 0)
    wkw = {"pipeline_mode": pl.Buffered(1)}
    return pl.pallas_call(
        _ffn_body,
        out_shape=jax.ShapeDtypeStruct((m_rows, d_out), out_dtype),
        grid=(m_rows // tile_m,),
        in_specs=[
            pl.BlockSpec((tile_m, d_in), lambda i: (i, 0)),
            pl.BlockSpec((d_in, d_mid), const, **wkw),
            pl.BlockSpec((1, d_mid), const, **wkw),
            pl.BlockSpec((d_mid, d_out), const, **wkw),
            pl.BlockSpec((1, d_out), const, **wkw),
        ],
        out_specs=pl.BlockSpec((tile_m, d_out), lambda i: (i, 0)),
        compiler_params=pltpu.CompilerParams(
            dimension_semantics=("parallel",),
            vmem_limit_bytes=60 * 1024 * 1024,
        ),
    )


@jax.jit
def kernel(x, w1, b1, w2, b2):
    B, S, H = x.shape
    FF = w1.shape[1]
    M = B * S
    x2 = x.reshape(M, H)

    w1b = w1.astype(jnp.bfloat16)
    w2b = w2.astype(jnp.bfloat16)
    b1f = b1.astype(jnp.float32).reshape(1, FF)
    b2f = b2.astype(jnp.float32).reshape(1, H)

    tile_m = _TILE_M
    while M % tile_m:
        tile_m //= 2

    out = _ffn_call(M, tile_m, H, FF, H, x.dtype)(x2, w1b, b1f, w2b, b2f)
    return out.reshape(B, S, H)
